# baseline (speedup 1.0000x reference)
import functools

import jax
import jax.numpy as jnp
from jax import lax
from jax.experimental import pallas as pl
from jax.experimental.pallas import tpu as pltpu
from jax.experimental.pallas import tpu_sc as plsc

N = 10000
E = 320000
D = 128
HH = 64
NP = 10240
NTILES = 16
ROWS_PER_TILE = NP // NTILES
CH = 128
NCH_FULL = 157
E_PT = NCH_FULL * CH
NCH_LAST = (E - (NTILES - 1) * E_PT) // CH
RB = 5
QB = 10
GA = 3
IA = 6
L = 16
RPC = ROWS_PER_TILE // CH

_SELU_ALPHA = 1.6732632423543772
_SELU_SCALE = 1.0507009873554805


def _edge_pass(ei, tb, ab, e0, nch, idx_s, idx_d, rows, g_sem, s_sem,
               i_sem, d_sem, hbm_tb=None):
    def issue_gather(jc, uc, bc, static_k=None):
        if hbm_tb is None:
            pltpu.async_copy(tb.at[idx_s.at[uc]], rows.at[bc], g_sem.at[bc])
            return
        if static_k is not None:
            src_tb = hbm_tb if static_k % 3 == 0 else tb
            pltpu.async_copy(src_tb.at[idx_s.at[uc]], rows.at[bc],
                             g_sem.at[bc])
            return

        @pl.when(lax.rem(jc, 3) == 0)
        def _():
            pltpu.async_copy(hbm_tb.at[idx_s.at[uc]], rows.at[bc],
                             g_sem.at[bc])

        @pl.when(lax.rem(jc, 3) != 0)
        def _():
            pltpu.async_copy(tb.at[idx_s.at[uc]], rows.at[bc], g_sem.at[bc])

    for k in range(IA):
        pltpu.async_copy(ei.at[0, pl.ds(e0 + k * CH, CH)], idx_s.at[k],
                         i_sem.at[k])
        pltpu.async_copy(ei.at[1, pl.ds(e0 + k * CH, CH)], idx_d.at[k],
                         d_sem.at[k])
    for k in range(GA):
        pltpu.make_async_copy(ei.at[0, pl.ds(0, CH)], idx_s.at[k],
                              i_sem.at[k]).wait()
        issue_gather(k, k, k, static_k=k)

    def step(j, carry):
        u = lax.rem(j, QB)
        b = lax.rem(j, RB)
        pltpu.make_async_copy(tb.at[idx_s.at[0]], rows.at[b],
                              g_sem.at[b]).wait()
        pltpu.make_async_copy(ei.at[1, pl.ds(0, CH)], idx_d.at[u],
                              d_sem.at[u]).wait()
        pltpu.async_copy(rows.at[b], ab.at[idx_d.at[u]], s_sem.at[b],
                         add=True)

        jp = j + GA

        @pl.when(jp < nch)
        def _():
            up = lax.rem(jp, QB)
            bp = lax.rem(jp, RB)

            @pl.when(jp >= RB)
            def _():
                pltpu.make_async_copy(rows.at[bp], ab.at[idx_d.at[up]],
                                      s_sem.at[bp]).wait()

            pltpu.make_async_copy(ei.at[0, pl.ds(0, CH)], idx_s.at[up],
                                  i_sem.at[up]).wait()
            issue_gather(jp, up, bp)

        ji = j + IA

        @pl.when(ji < nch)
        def _():
            ui = lax.rem(ji, QB)
            pltpu.async_copy(ei.at[0, pl.ds(e0 + ji * CH, CH)],
                             idx_s.at[ui], i_sem.at[ui])
            pltpu.async_copy(ei.at[1, pl.ds(e0 + ji * CH, CH)],
                             idx_d.at[ui], d_sem.at[ui])
        return carry

    lax.fori_loop(0, nch, step, 0)

    for i in range(RB):
        k = nch - RB + i
        pltpu.make_async_copy(rows.at[lax.rem(k, RB)],
                              ab.at[idx_d.at[lax.rem(k, QB)]],
                              s_sem.at[lax.rem(k, RB)]).wait()


@functools.partial(
    pl.kernel,
    mesh=plsc.VectorSubcoreMesh(core_axis_name="c", subcore_axis_name="s"),
    out_type=(
        jax.ShapeDtypeStruct((NP, HH), jnp.float32),
        jax.ShapeDtypeStruct((NP, HH), jnp.float32),
    ),
    scratch_types=[
        pltpu.VMEM_SHARED((NP, HH), jnp.float32),
        pltpu.VMEM_SHARED((NP, HH), jnp.float32),
        pltpu.VMEM((QB, CH), jnp.int32),
        pltpu.VMEM((QB, CH), jnp.int32),
        pltpu.VMEM((RB, CH, HH), jnp.float32),
        pltpu.VMEM((HH,), jnp.float32),
        pltpu.SemaphoreType.DMA((RB,)),
        pltpu.SemaphoreType.DMA((RB,)),
        pltpu.SemaphoreType.DMA((QB,)),
        pltpu.SemaphoreType.DMA((QB,)),
    ],
    compiler_params=pltpu.CompilerParams(use_tc_tiling_on_sc=False),
)
def _agg123(ya, yb, ei, b1, oa, ob, bufa, bufb, idx_s, idx_d, rows, b1v,
            g_sem, s_sem, i_sem, d_sem):
    c = lax.axis_index("c")
    s = lax.axis_index("s")
    r0 = s * ROWS_PER_TILE

    def stage(table):
        pltpu.sync_copy(table.at[pl.ds(r0, ROWS_PER_TILE)],
                        bufa.at[pl.ds(r0, ROWS_PER_TILE)])
        pltpu.sync_copy(table.at[pl.ds(r0, ROWS_PER_TILE)],
                        bufb.at[pl.ds(r0, ROWS_PER_TILE)])

    @pl.when(c == 0)
    def _():
        stage(ya)

    @pl.when(c == 1)
    def _():
        stage(yb)

    pltpu.sync_copy(b1.at[pl.ds(c * HH, HH)], b1v)
    last = NTILES - 1
    nch = jnp.where(s == last, NCH_LAST, NCH_FULL)
    e0 = s * E_PT
    plsc.subcore_barrier()

    @pl.when(c == 0)
    def _():
        _edge_pass(ei, bufa, bufb, e0, nch, idx_s, idx_d, rows, g_sem,
                   s_sem, i_sem, d_sem, hbm_tb=ya)

    @pl.when(c == 1)
    def _():
        _edge_pass(ei, bufa, bufb, e0, nch, idx_s, idx_d, rows, g_sem,
                   s_sem, i_sem, d_sem, hbm_tb=yb)

    plsc.subcore_barrier()
    pltpu.async_copy(bufb.at[pl.ds(r0, CH)], rows.at[0], g_sem.at[0])
    for k in range(RPC):
        rr = r0 + k * CH
        if k >= 1:
            pltpu.make_async_copy(rows.at[(k - 1) % 2],
                                  bufa.at[pl.ds(rr - CH, CH)],
                                  s_sem.at[(k - 1) % 2]).wait()
        pltpu.make_async_copy(bufb.at[pl.ds(rr, CH)], rows.at[k % 2],
                              g_sem.at[k % 2]).wait()
        if k + 1 < RPC:
            pltpu.async_copy(bufb.at[pl.ds(rr + CH, CH)],
                             rows.at[(k + 1) % 2], g_sem.at[(k + 1) % 2])
        pltpu.async_copy(rows.at[k % 2], bufa.at[pl.ds(rr, CH)],
                         s_sem.at[k % 2])
    pltpu.make_async_copy(rows.at[(RPC - 1) % 2],
                          bufa.at[pl.ds(r0, CH)],
                          s_sem.at[(RPC - 1) % 2]).wait()
    plsc.subcore_barrier()
    _edge_pass(ei, bufb, bufa, e0, nch, idx_s, idx_d, rows, g_sem, s_sem,
               i_sem, d_sem)
    plsc.subcore_barrier()
    pltpu.async_copy(bufa.at[pl.ds(r0, CH)], rows.at[0], g_sem.at[0])
    for k in range(RPC):
        rr = r0 + k * CH
        if k >= 1:
            prr = rr - CH
            pltpu.make_async_copy(rows.at[(k - 1) % 2],
                                  bufa.at[pl.ds(prr, CH)],
                                  s_sem.at[(k - 1) % 2]).wait()
            pltpu.make_async_copy(rows.at[(k - 1) % 2],
                                  bufb.at[pl.ds(prr, CH)],
                                  s_sem.at[(k - 1) % 2]).wait()
        pltpu.make_async_copy(bufa.at[pl.ds(rr, CH)], rows.at[k % 2],
                              g_sem.at[k % 2]).wait()
        if k + 1 < RPC:
            pltpu.async_copy(bufa.at[pl.ds(rr + CH, CH)],
                             rows.at[(k + 1) % 2], g_sem.at[(k + 1) % 2])

        def selu_rows(r4, carry):
            for dr in range(4):
                for q in range(HH // L):
                    v = (rows[k % 2, r4 * 4 + dr, pl.ds(q * L, L)]
                         + b1v[pl.ds(q * L, L)])
                    ev = _SELU_ALPHA * jnp.exp(v) - _SELU_ALPHA
                    rows[k % 2, r4 * 4 + dr, pl.ds(q * L, L)] = (
                        _SELU_SCALE * jnp.where(v > 0, v, ev))
            return carry

        lax.fori_loop(0, CH // 4, selu_rows, 0)
        pltpu.async_copy(rows.at[k % 2], bufa.at[pl.ds(rr, CH)],
                         s_sem.at[k % 2])
        pltpu.async_copy(rows.at[k % 2], bufb.at[pl.ds(rr, CH)],
                         s_sem.at[k % 2])
    pltpu.make_async_copy(rows.at[(RPC - 1) % 2], bufa.at[pl.ds(r0, CH)],
                          s_sem.at[(RPC - 1) % 2]).wait()
    pltpu.make_async_copy(rows.at[(RPC - 1) % 2], bufb.at[pl.ds(r0, CH)],
                          s_sem.at[(RPC - 1) % 2]).wait()
    plsc.subcore_barrier()
    _edge_pass(ei, bufa, bufb, e0, nch, idx_s, idx_d, rows, g_sem, s_sem,
               i_sem, d_sem)
    plsc.subcore_barrier()

    @pl.when(c == 0)
    def _():
        pltpu.sync_copy(bufb.at[pl.ds(r0, ROWS_PER_TILE)],
                        oa.at[pl.ds(r0, ROWS_PER_TILE)])

    @pl.when(c == 1)
    def _():
        pltpu.sync_copy(bufb.at[pl.ds(r0, ROWS_PER_TILE)],
                        ob.at[pl.ds(r0, ROWS_PER_TILE)])


def _lin1_body(x_ref, w1_ref, ya_ref, yb_ref):
    z = jnp.dot(x_ref[...], w1_ref[...], preferred_element_type=jnp.float32)
    ya_ref[...] = z[:, :HH]
    yb_ref[...] = z[:, HH:]


def _lin1(x, w1):
    br = ROWS_PER_TILE
    grid = (NP // br,)
    return pl.pallas_call(
        _lin1_body,
        grid=grid,
        in_specs=[
            pl.BlockSpec((br, D), lambda i: (i, 0)),
            pl.BlockSpec((D, D), lambda i: (0, 0)),
        ],
        out_specs=[
            pl.BlockSpec((br, HH), lambda i: (i, 0)),
            pl.BlockSpec((br, HH), lambda i: (i, 0)),
        ],
        out_shape=[
            jax.ShapeDtypeStruct((NP, HH), jnp.float32),
            jax.ShapeDtypeStruct((NP, HH), jnp.float32),
        ],
    )(x, w1)


def _out_body(qa_ref, qb_ref, w2_ref, b2_ref, o_ref):
    z = (jnp.dot(qa_ref[...], w2_ref[:HH, :],
                 preferred_element_type=jnp.float32)
         + jnp.dot(qb_ref[...], w2_ref[HH:, :],
                   preferred_element_type=jnp.float32)
         + b2_ref[...])
    m = jnp.max(z, axis=1, keepdims=True)
    lse = jnp.log(jnp.sum(jnp.exp(z - m), axis=1, keepdims=True)) + m
    o_ref[...] = z - lse


def _outk(qa, qb, w2, b2):
    br = 1000
    grid = (N // br,)
    return pl.pallas_call(
        _out_body,
        grid=grid,
        in_specs=[
            pl.BlockSpec((br, HH), lambda i: (i, 0)),
            pl.BlockSpec((br, HH), lambda i: (i, 0)),
            pl.BlockSpec((D, D), lambda i: (0, 0)),
            pl.BlockSpec((1, D), lambda i: (0, 0)),
        ],
        out_specs=pl.BlockSpec((br, D), lambda i: (i, 0)),
        out_shape=jax.ShapeDtypeStruct((N, D), jnp.float32),
    )(qa, qb, w2, b2)


def kernel(x, edge_index, W1, b1, W2, b2):
    ya, yb = _lin1(x, W1)
    h4a, h4b = _agg123(ya, yb, edge_index, b1)
    return _outk(h4a, h4b, W2, b2.reshape(1, D))


# speedup vs baseline: 1.1450x; 1.1450x over previous
import functools

import jax
import jax.numpy as jnp
from jax import lax
from jax.experimental import pallas as pl
from jax.experimental.pallas import tpu as pltpu
from jax.experimental.pallas import tpu_sc as plsc

N = 10000
E = 320000
D = 128
HH = 64
NP = 10240
NTILES = 16
ROWS_PER_TILE = NP // NTILES
CH = 128
NCH_FULL = 157
E_PT = NCH_FULL * CH
NCH_LAST = (E - (NTILES - 1) * E_PT) // CH
RB = 5
QB = 10
GA = 3
IA = 6
L = 16
RPC = ROWS_PER_TILE // CH

_SELU_ALPHA = 1.6732632423543772
_SELU_SCALE = 1.0507009873554805


def _edge_pass(ei, tb, ab, e0, nch, idx_s, idx_d, rows, g_sem, s_sem,
               i_sem, d_sem, hbm_tb=None):
    def issue_gather(jc, uc, bc, static_k=None):
        if hbm_tb is None:
            pltpu.async_copy(tb.at[idx_s.at[uc]], rows.at[bc], g_sem.at[bc])
            return
        if static_k is not None:
            src_tb = hbm_tb if static_k % 3 == 0 else tb
            pltpu.async_copy(src_tb.at[idx_s.at[uc]], rows.at[bc],
                             g_sem.at[bc])
            return

        @pl.when(lax.rem(jc, 3) == 0)
        def _():
            pltpu.async_copy(hbm_tb.at[idx_s.at[uc]], rows.at[bc],
                             g_sem.at[bc])

        @pl.when(lax.rem(jc, 3) != 0)
        def _():
            pltpu.async_copy(tb.at[idx_s.at[uc]], rows.at[bc], g_sem.at[bc])

    for k in range(IA):
        pltpu.async_copy(ei.at[0, pl.ds(e0 + k * CH, CH)], idx_s.at[k],
                         i_sem.at[k])
        pltpu.async_copy(ei.at[1, pl.ds(e0 + k * CH, CH)], idx_d.at[k],
                         d_sem.at[k])
    for k in range(GA):
        pltpu.make_async_copy(ei.at[0, pl.ds(0, CH)], idx_s.at[k],
                              i_sem.at[k]).wait()
        issue_gather(k, k, k, static_k=k)

    def step(j, carry):
        u = lax.rem(j, QB)
        b = lax.rem(j, RB)
        pltpu.make_async_copy(tb.at[idx_s.at[0]], rows.at[b],
                              g_sem.at[b]).wait()
        pltpu.make_async_copy(ei.at[1, pl.ds(0, CH)], idx_d.at[u],
                              d_sem.at[u]).wait()
        pltpu.async_copy(rows.at[b], ab.at[idx_d.at[u]], s_sem.at[b],
                         add=True)

        jp = j + GA

        @pl.when(jp < nch)
        def _():
            up = lax.rem(jp, QB)
            bp = lax.rem(jp, RB)

            @pl.when(jp >= RB)
            def _():
                pltpu.make_async_copy(rows.at[bp], ab.at[idx_d.at[up]],
                                      s_sem.at[bp]).wait()

            pltpu.make_async_copy(ei.at[0, pl.ds(0, CH)], idx_s.at[up],
                                  i_sem.at[up]).wait()
            issue_gather(jp, up, bp)

        ji = j + IA

        @pl.when(ji < nch)
        def _():
            ui = lax.rem(ji, QB)
            pltpu.async_copy(ei.at[0, pl.ds(e0 + ji * CH, CH)],
                             idx_s.at[ui], i_sem.at[ui])
            pltpu.async_copy(ei.at[1, pl.ds(e0 + ji * CH, CH)],
                             idx_d.at[ui], d_sem.at[ui])
        return carry

    lax.fori_loop(0, nch, step, 0)

    for i in range(RB):
        k = nch - RB + i
        pltpu.make_async_copy(rows.at[lax.rem(k, RB)],
                              ab.at[idx_d.at[lax.rem(k, QB)]],
                              s_sem.at[lax.rem(k, RB)]).wait()


@functools.partial(
    pl.kernel,
    mesh=plsc.VectorSubcoreMesh(core_axis_name="c", subcore_axis_name="s"),
    out_type=(
        jax.ShapeDtypeStruct((NP, HH), jnp.float32),
        jax.ShapeDtypeStruct((NP, HH), jnp.float32),
    ),
    scratch_types=[
        pltpu.VMEM_SHARED((NP, HH), jnp.float32),
        pltpu.VMEM_SHARED((NP, HH), jnp.float32),
        pltpu.VMEM((QB, CH), jnp.int32),
        pltpu.VMEM((QB, CH), jnp.int32),
        pltpu.VMEM((RB, CH, HH), jnp.float32),
        pltpu.VMEM((HH,), jnp.float32),
        pltpu.SemaphoreType.DMA((RB,)),
        pltpu.SemaphoreType.DMA((RB,)),
        pltpu.SemaphoreType.DMA((QB,)),
        pltpu.SemaphoreType.DMA((QB,)),
    ],
    compiler_params=pltpu.CompilerParams(use_tc_tiling_on_sc=False),
)
def _agg123(ya, yb, ei, b1, oa, ob, bufa, bufb, idx_s, idx_d, rows, b1v,
            g_sem, s_sem, i_sem, d_sem):
    c = lax.axis_index("c")
    s = lax.axis_index("s")
    r0 = s * ROWS_PER_TILE

    def stage(table):
        pltpu.sync_copy(table.at[pl.ds(r0, ROWS_PER_TILE)],
                        bufa.at[pl.ds(r0, ROWS_PER_TILE)])
        pltpu.sync_copy(table.at[pl.ds(r0, ROWS_PER_TILE)],
                        bufb.at[pl.ds(r0, ROWS_PER_TILE)])

    @pl.when(c == 0)
    def _():
        stage(ya)

    @pl.when(c == 1)
    def _():
        stage(yb)

    pltpu.sync_copy(b1.at[pl.ds(c * HH, HH)], b1v)
    last = NTILES - 1
    nch = jnp.where(s == last, NCH_LAST, NCH_FULL)
    e0 = s * E_PT
    plsc.subcore_barrier()

    _edge_pass(ei, bufa, bufb, e0, nch, idx_s, idx_d, rows, g_sem,
               s_sem, i_sem, d_sem)
    plsc.subcore_barrier()
    pltpu.async_copy(bufb.at[pl.ds(r0, CH)], rows.at[0], g_sem.at[0])
    for k in range(RPC):
        rr = r0 + k * CH
        if k >= 1:
            pltpu.make_async_copy(rows.at[(k - 1) % 2],
                                  bufa.at[pl.ds(rr - CH, CH)],
                                  s_sem.at[(k - 1) % 2]).wait()
        pltpu.make_async_copy(bufb.at[pl.ds(rr, CH)], rows.at[k % 2],
                              g_sem.at[k % 2]).wait()
        if k + 1 < RPC:
            pltpu.async_copy(bufb.at[pl.ds(rr + CH, CH)],
                             rows.at[(k + 1) % 2], g_sem.at[(k + 1) % 2])
        pltpu.async_copy(rows.at[k % 2], bufa.at[pl.ds(rr, CH)],
                         s_sem.at[k % 2])
    pltpu.make_async_copy(rows.at[(RPC - 1) % 2],
                          bufa.at[pl.ds(r0, CH)],
                          s_sem.at[(RPC - 1) % 2]).wait()
    plsc.subcore_barrier()
    _edge_pass(ei, bufb, bufa, e0, nch, idx_s, idx_d, rows, g_sem, s_sem,
               i_sem, d_sem)
    plsc.subcore_barrier()
    pltpu.async_copy(bufa.at[pl.ds(r0, CH)], rows.at[0], g_sem.at[0])
    for k in range(RPC):
        rr = r0 + k * CH
        if k >= 1:
            prr = rr - CH
            pltpu.make_async_copy(rows.at[(k - 1) % 2],
                                  bufa.at[pl.ds(prr, CH)],
                                  s_sem.at[(k - 1) % 2]).wait()
            pltpu.make_async_copy(rows.at[(k - 1) % 2],
                                  bufb.at[pl.ds(prr, CH)],
                                  s_sem.at[(k - 1) % 2]).wait()
        pltpu.make_async_copy(bufa.at[pl.ds(rr, CH)], rows.at[k % 2],
                              g_sem.at[k % 2]).wait()
        if k + 1 < RPC:
            pltpu.async_copy(bufa.at[pl.ds(rr + CH, CH)],
                             rows.at[(k + 1) % 2], g_sem.at[(k + 1) % 2])

        def selu_rows(r4, carry):
            for dr in range(4):
                for q in range(HH // L):
                    v = (rows[k % 2, r4 * 4 + dr, pl.ds(q * L, L)]
                         + b1v[pl.ds(q * L, L)])
                    ev = _SELU_ALPHA * jnp.exp(v) - _SELU_ALPHA
                    rows[k % 2, r4 * 4 + dr, pl.ds(q * L, L)] = (
                        _SELU_SCALE * jnp.where(v > 0, v, ev))
            return carry

        lax.fori_loop(0, CH // 4, selu_rows, 0)
        pltpu.async_copy(rows.at[k % 2], bufa.at[pl.ds(rr, CH)],
                         s_sem.at[k % 2])
        pltpu.async_copy(rows.at[k % 2], bufb.at[pl.ds(rr, CH)],
                         s_sem.at[k % 2])
    pltpu.make_async_copy(rows.at[(RPC - 1) % 2], bufa.at[pl.ds(r0, CH)],
                          s_sem.at[(RPC - 1) % 2]).wait()
    pltpu.make_async_copy(rows.at[(RPC - 1) % 2], bufb.at[pl.ds(r0, CH)],
                          s_sem.at[(RPC - 1) % 2]).wait()
    plsc.subcore_barrier()
    _edge_pass(ei, bufa, bufb, e0, nch, idx_s, idx_d, rows, g_sem, s_sem,
               i_sem, d_sem)
    plsc.subcore_barrier()

    @pl.when(c == 0)
    def _():
        pltpu.sync_copy(bufb.at[pl.ds(r0, ROWS_PER_TILE)],
                        oa.at[pl.ds(r0, ROWS_PER_TILE)])

    @pl.when(c == 1)
    def _():
        pltpu.sync_copy(bufb.at[pl.ds(r0, ROWS_PER_TILE)],
                        ob.at[pl.ds(r0, ROWS_PER_TILE)])


def _lin1_body(x_ref, w1_ref, ya_ref, yb_ref):
    z = jnp.dot(x_ref[...], w1_ref[...], preferred_element_type=jnp.float32)
    ya_ref[...] = z[:, :HH]
    yb_ref[...] = z[:, HH:]


def _lin1(x, w1):
    br = ROWS_PER_TILE
    grid = (NP // br,)
    return pl.pallas_call(
        _lin1_body,
        grid=grid,
        in_specs=[
            pl.BlockSpec((br, D), lambda i: (i, 0)),
            pl.BlockSpec((D, D), lambda i: (0, 0)),
        ],
        out_specs=[
            pl.BlockSpec((br, HH), lambda i: (i, 0)),
            pl.BlockSpec((br, HH), lambda i: (i, 0)),
        ],
        out_shape=[
            jax.ShapeDtypeStruct((NP, HH), jnp.float32),
            jax.ShapeDtypeStruct((NP, HH), jnp.float32),
        ],
    )(x, w1)


def _out_body(qa_ref, qb_ref, w2_ref, b2_ref, o_ref):
    z = (jnp.dot(qa_ref[...], w2_ref[:HH, :],
                 preferred_element_type=jnp.float32)
         + jnp.dot(qb_ref[...], w2_ref[HH:, :],
                   preferred_element_type=jnp.float32)
         + b2_ref[...])
    m = jnp.max(z, axis=1, keepdims=True)
    lse = jnp.log(jnp.sum(jnp.exp(z - m), axis=1, keepdims=True)) + m
    o_ref[...] = z - lse


def _outk(qa, qb, w2, b2):
    br = 1000
    grid = (N // br,)
    return pl.pallas_call(
        _out_body,
        grid=grid,
        in_specs=[
            pl.BlockSpec((br, HH), lambda i: (i, 0)),
            pl.BlockSpec((br, HH), lambda i: (i, 0)),
            pl.BlockSpec((D, D), lambda i: (0, 0)),
            pl.BlockSpec((1, D), lambda i: (0, 0)),
        ],
        out_specs=pl.BlockSpec((br, D), lambda i: (i, 0)),
        out_shape=jax.ShapeDtypeStruct((N, D), jnp.float32),
    )(qa, qb, w2, b2)


def kernel(x, edge_index, W1, b1, W2, b2):
    ya, yb = _lin1(x, W1)
    h4a, h4b = _agg123(ya, yb, edge_index, b1)
    return _outk(h4a, h4b, W2, b2.reshape(1, D))


# speedup vs baseline: 1.2151x; 1.0612x over previous
import functools

import jax
import jax.numpy as jnp
from jax import lax
from jax.experimental import pallas as pl
from jax.experimental.pallas import tpu as pltpu
from jax.experimental.pallas import tpu_sc as plsc

N = 10000
E = 320000
D = 128
HH = 64
NP = 10240
NTILES = 16
ROWS_PER_TILE = NP // NTILES
LAST_ROWS = N - (NTILES - 1) * ROWS_PER_TILE
CH = 128
NCH_FULL = 157
E_PT = NCH_FULL * CH
NCH_LAST = (E - (NTILES - 1) * E_PT) // CH
RB = 4
QB = 8
GA = 2
IA = 6

_SELU_ALPHA = 1.6732632423543772
_SELU_SCALE = 1.0507009873554805

_SC_SCRATCH = [
    pltpu.VMEM_SHARED((NP, HH), jnp.float32),
    pltpu.VMEM_SHARED((NP, HH), jnp.float32),
    pltpu.VMEM((QB, CH), jnp.int32),
    pltpu.VMEM((QB, CH), jnp.int32),
    pltpu.VMEM((RB, CH, HH), jnp.float32),
    pltpu.SemaphoreType.DMA((RB,)),
    pltpu.SemaphoreType.DMA((RB,)),
    pltpu.SemaphoreType.DMA((QB,)),
    pltpu.SemaphoreType.DMA((QB,)),
]


def _edge_pass(ei, tb, ab, e0, nch, idx_s, idx_d, rows, g_sem, s_sem,
               i_sem, d_sem):
    for k in range(IA):
        pltpu.async_copy(ei.at[0, pl.ds(e0 + k * CH, CH)], idx_s.at[k],
                         i_sem.at[k])
        pltpu.async_copy(ei.at[1, pl.ds(e0 + k * CH, CH)], idx_d.at[k],
                         d_sem.at[k])
    for k in range(GA):
        pltpu.make_async_copy(ei.at[0, pl.ds(0, CH)], idx_s.at[k],
                              i_sem.at[k]).wait()
        pltpu.async_copy(tb.at[idx_s.at[k]], rows.at[k], g_sem.at[k])

    def step(j, carry):
        u = lax.rem(j, QB)
        b = lax.rem(j, RB)
        pltpu.make_async_copy(tb.at[idx_s.at[0]], rows.at[b],
                              g_sem.at[b]).wait()
        pltpu.make_async_copy(ei.at[1, pl.ds(0, CH)], idx_d.at[u],
                              d_sem.at[u]).wait()
        pltpu.async_copy(rows.at[b], ab.at[idx_d.at[u]], s_sem.at[b],
                         add=True)

        jp = j + GA

        @pl.when(jp < nch)
        def _():
            up = lax.rem(jp, QB)
            bp = lax.rem(jp, RB)

            @pl.when(jp >= RB)
            def _():
                pltpu.make_async_copy(rows.at[bp], ab.at[idx_d.at[up]],
                                      s_sem.at[bp]).wait()

            pltpu.make_async_copy(ei.at[0, pl.ds(0, CH)], idx_s.at[up],
                                  i_sem.at[up]).wait()
            pltpu.async_copy(tb.at[idx_s.at[up]], rows.at[bp], g_sem.at[bp])

        ji = j + IA

        @pl.when(ji < nch)
        def _():
            ui = lax.rem(ji, QB)
            pltpu.async_copy(ei.at[0, pl.ds(e0 + ji * CH, CH)],
                             idx_s.at[ui], i_sem.at[ui])
            pltpu.async_copy(ei.at[1, pl.ds(e0 + ji * CH, CH)],
                             idx_d.at[ui], d_sem.at[ui])
        return carry

    lax.fori_loop(0, nch, step, 0)

    for i in range(RB):
        k = nch - RB + i
        pltpu.make_async_copy(rows.at[lax.rem(k, RB)],
                              ab.at[idx_d.at[lax.rem(k, QB)]],
                              s_sem.at[lax.rem(k, RB)]).wait()


@functools.partial(
    pl.kernel,
    mesh=plsc.VectorSubcoreMesh(core_axis_name="c", subcore_axis_name="s"),
    out_type=(
        jax.ShapeDtypeStruct((NP, HH), jnp.float32),
        jax.ShapeDtypeStruct((NP, HH), jnp.float32),
    ),
    scratch_types=list(_SC_SCRATCH),
    compiler_params=pltpu.CompilerParams(use_tc_tiling_on_sc=False),
)
def _agg12(x, ei, oa, ob, bufa, bufb, idx_s, idx_d, rows, g_sem, s_sem,
           i_sem, d_sem):
    c = lax.axis_index("c")
    s = lax.axis_index("s")
    r0 = s * ROWS_PER_TILE
    col = c * HH
    last = NTILES - 1

    @pl.when(s < last)
    def _():
        pltpu.sync_copy(x.at[pl.ds(r0, ROWS_PER_TILE), pl.ds(col, HH)],
                        bufa.at[pl.ds(r0, ROWS_PER_TILE)])
        pltpu.sync_copy(x.at[pl.ds(r0, ROWS_PER_TILE), pl.ds(col, HH)],
                        bufb.at[pl.ds(r0, ROWS_PER_TILE)])

    @pl.when(s == last)
    def _():
        pltpu.sync_copy(x.at[pl.ds(r0, LAST_ROWS), pl.ds(col, HH)],
                        bufa.at[pl.ds(r0, LAST_ROWS)])
        pltpu.sync_copy(x.at[pl.ds(r0, LAST_ROWS), pl.ds(col, HH)],
                        bufb.at[pl.ds(r0, LAST_ROWS)])

    nch = jnp.where(s == last, NCH_LAST, NCH_FULL)
    e0 = s * E_PT
    plsc.subcore_barrier()
    _edge_pass(ei, bufa, bufb, e0, nch, idx_s, idx_d, rows, g_sem, s_sem,
               i_sem, d_sem)
    plsc.subcore_barrier()
    for k in range(ROWS_PER_TILE // CH):
        rr = r0 + k * CH
        pltpu.sync_copy(bufb.at[pl.ds(rr, CH)], rows.at[0])
        pltpu.sync_copy(rows.at[0], bufa.at[pl.ds(rr, CH)])
    plsc.subcore_barrier()
    _edge_pass(ei, bufb, bufa, e0, nch, idx_s, idx_d, rows, g_sem, s_sem,
               i_sem, d_sem)
    plsc.subcore_barrier()

    @pl.when(c == 0)
    def _():
        pltpu.sync_copy(bufa.at[pl.ds(r0, ROWS_PER_TILE)],
                        oa.at[pl.ds(r0, ROWS_PER_TILE)])

    @pl.when(c == 1)
    def _():
        pltpu.sync_copy(bufa.at[pl.ds(r0, ROWS_PER_TILE)],
                        ob.at[pl.ds(r0, ROWS_PER_TILE)])


@functools.partial(
    pl.kernel,
    mesh=plsc.VectorSubcoreMesh(core_axis_name="c", subcore_axis_name="s"),
    out_type=(
        jax.ShapeDtypeStruct((NP, HH), jnp.float32),
        jax.ShapeDtypeStruct((NP, HH), jnp.float32),
    ),
    scratch_types=list(_SC_SCRATCH),
    compiler_params=pltpu.CompilerParams(use_tc_tiling_on_sc=False),
)
def _agg3(ha, hb, ei, oa, ob, bufa, bufb, idx_s, idx_d, rows, g_sem, s_sem,
          i_sem, d_sem):
    c = lax.axis_index("c")
    s = lax.axis_index("s")
    r0 = s * ROWS_PER_TILE
    last = NTILES - 1

    def stage(table):
        pltpu.sync_copy(table.at[pl.ds(r0, ROWS_PER_TILE)],
                        bufa.at[pl.ds(r0, ROWS_PER_TILE)])
        pltpu.sync_copy(table.at[pl.ds(r0, ROWS_PER_TILE)],
                        bufb.at[pl.ds(r0, ROWS_PER_TILE)])

    @pl.when(c == 0)
    def _():
        stage(ha)

    @pl.when(c == 1)
    def _():
        stage(hb)

    nch = jnp.where(s == last, NCH_LAST, NCH_FULL)
    e0 = s * E_PT
    plsc.subcore_barrier()
    _edge_pass(ei, bufa, bufb, e0, nch, idx_s, idx_d, rows, g_sem, s_sem,
               i_sem, d_sem)
    plsc.subcore_barrier()

    @pl.when(c == 0)
    def _():
        pltpu.sync_copy(bufb.at[pl.ds(r0, ROWS_PER_TILE)],
                        oa.at[pl.ds(r0, ROWS_PER_TILE)])

    @pl.when(c == 1)
    def _():
        pltpu.sync_copy(bufb.at[pl.ds(r0, ROWS_PER_TILE)],
                        ob.at[pl.ds(r0, ROWS_PER_TILE)])


def _mlp_body(oa_ref, ob_ref, w1_ref, b1_ref, pa_ref, pb_ref):
    z = (jnp.dot(oa_ref[...], w1_ref[:HH, :],
                 preferred_element_type=jnp.float32)
         + jnp.dot(ob_ref[...], w1_ref[HH:, :],
                   preferred_element_type=jnp.float32)
         + b1_ref[...])
    act = _SELU_SCALE * jnp.where(z > 0, z, _SELU_ALPHA * (jnp.exp(z) - 1.0))
    pa_ref[...] = act[:, :HH]
    pb_ref[...] = act[:, HH:]


def _mlp(oa, ob, w1, b1):
    br = 1024
    grid = (NP // br,)
    return pl.pallas_call(
        _mlp_body,
        grid=grid,
        in_specs=[
            pl.BlockSpec((br, HH), lambda i: (i, 0)),
            pl.BlockSpec((br, HH), lambda i: (i, 0)),
            pl.BlockSpec((D, D), lambda i: (0, 0)),
            pl.BlockSpec((1, D), lambda i: (0, 0)),
        ],
        out_specs=[
            pl.BlockSpec((br, HH), lambda i: (i, 0)),
            pl.BlockSpec((br, HH), lambda i: (i, 0)),
        ],
        out_shape=[
            jax.ShapeDtypeStruct((NP, HH), jnp.float32),
            jax.ShapeDtypeStruct((NP, HH), jnp.float32),
        ],
    )(oa, ob, w1, b1)


def _out_body(qa_ref, qb_ref, w2_ref, b2_ref, o_ref):
    z = (jnp.dot(qa_ref[...], w2_ref[:HH, :],
                 preferred_element_type=jnp.float32)
         + jnp.dot(qb_ref[...], w2_ref[HH:, :],
                   preferred_element_type=jnp.float32)
         + b2_ref[...])
    m = jnp.max(z, axis=1, keepdims=True)
    lse = jnp.log(jnp.sum(jnp.exp(z - m), axis=1, keepdims=True)) + m
    o_ref[...] = z - lse


def _outk(qa, qb, w2, b2):
    br = 1000
    grid = (N // br,)
    return pl.pallas_call(
        _out_body,
        grid=grid,
        in_specs=[
            pl.BlockSpec((br, HH), lambda i: (i, 0)),
            pl.BlockSpec((br, HH), lambda i: (i, 0)),
            pl.BlockSpec((D, D), lambda i: (0, 0)),
            pl.BlockSpec((1, D), lambda i: (0, 0)),
        ],
        out_specs=pl.BlockSpec((br, D), lambda i: (i, 0)),
        out_shape=jax.ShapeDtypeStruct((N, D), jnp.float32),
    )(qa, qb, w2, b2)


def kernel(x, edge_index, W1, b1, W2, b2):
    h2a, h2b = _agg12(x, edge_index)
    h3a, h3b = _mlp(h2a, h2b, W1, b1.reshape(1, D))
    h4a, h4b = _agg3(h3a, h3b, edge_index)
    return _outk(h4a, h4b, W2, b2.reshape(1, D))


# speedup vs baseline: 1.2313x; 1.0133x over previous
import functools

import jax
import jax.numpy as jnp
from jax import lax
from jax.experimental import pallas as pl
from jax.experimental.pallas import tpu as pltpu
from jax.experimental.pallas import tpu_sc as plsc

N = 10000
E = 320000
D = 128
HH = 64
NP = 10240
NTILES = 16
ROWS_PER_TILE = NP // NTILES
LAST_ROWS = N - (NTILES - 1) * ROWS_PER_TILE
CH = 128
NCH_FULL = 157
E_PT = NCH_FULL * CH
NCH_LAST = (E - (NTILES - 1) * E_PT) // CH
RB = 4
QB = 8
GA = 2
IA = 6

_SELU_ALPHA = 1.6732632423543772
_SELU_SCALE = 1.0507009873554805

_SC_SCRATCH = [
    pltpu.VMEM_SHARED((NP, HH), jnp.float32),
    pltpu.VMEM_SHARED((NP, HH), jnp.float32),
    pltpu.VMEM((QB, CH), jnp.int32),
    pltpu.VMEM((QB, CH), jnp.int32),
    pltpu.VMEM((RB, CH, HH), jnp.float32),
    pltpu.SemaphoreType.DMA((RB,)),
    pltpu.SemaphoreType.DMA((RB,)),
    pltpu.SemaphoreType.DMA((QB,)),
    pltpu.SemaphoreType.DMA((QB,)),
]


def _edge_pass(ei, tb, ab, e0, nch, idx_s, idx_d, rows, g_sem, s_sem,
               i_sem, d_sem):
    for k in range(IA):
        pltpu.async_copy(ei.at[0, pl.ds(e0 + k * CH, CH)], idx_s.at[k],
                         i_sem.at[k])
        pltpu.async_copy(ei.at[1, pl.ds(e0 + k * CH, CH)], idx_d.at[k],
                         d_sem.at[k])
    for k in range(GA):
        pltpu.make_async_copy(ei.at[0, pl.ds(0, CH)], idx_s.at[k],
                              i_sem.at[k]).wait()
        pltpu.async_copy(tb.at[idx_s.at[k]], rows.at[k], g_sem.at[k])

    def step(j, carry):
        u = lax.rem(j, QB)
        b = lax.rem(j, RB)
        pltpu.make_async_copy(tb.at[idx_s.at[0]], rows.at[b],
                              g_sem.at[b]).wait()
        pltpu.make_async_copy(ei.at[1, pl.ds(0, CH)], idx_d.at[u],
                              d_sem.at[u]).wait()
        pltpu.async_copy(rows.at[b], ab.at[idx_d.at[u]], s_sem.at[b],
                         add=True)

        jp = j + GA

        @pl.when(jp < nch)
        def _():
            up = lax.rem(jp, QB)
            bp = lax.rem(jp, RB)

            @pl.when(jp >= RB)
            def _():
                pltpu.make_async_copy(rows.at[bp], ab.at[idx_d.at[up]],
                                      s_sem.at[bp]).wait()

            pltpu.make_async_copy(ei.at[0, pl.ds(0, CH)], idx_s.at[up],
                                  i_sem.at[up]).wait()
            pltpu.async_copy(tb.at[idx_s.at[up]], rows.at[bp], g_sem.at[bp])

        ji = j + IA

        @pl.when(ji < nch)
        def _():
            ui = lax.rem(ji, QB)
            pltpu.async_copy(ei.at[0, pl.ds(e0 + ji * CH, CH)],
                             idx_s.at[ui], i_sem.at[ui])
            pltpu.async_copy(ei.at[1, pl.ds(e0 + ji * CH, CH)],
                             idx_d.at[ui], d_sem.at[ui])
        return carry

    lax.fori_loop(0, nch, step, 0)

    for i in range(RB):
        k = nch - RB + i
        pltpu.make_async_copy(rows.at[lax.rem(k, RB)],
                              ab.at[idx_d.at[lax.rem(k, QB)]],
                              s_sem.at[lax.rem(k, RB)]).wait()


@functools.partial(
    pl.kernel,
    mesh=plsc.VectorSubcoreMesh(core_axis_name="c", subcore_axis_name="s"),
    out_type=(
        jax.ShapeDtypeStruct((NP, HH), jnp.float32),
        jax.ShapeDtypeStruct((NP, HH), jnp.float32),
    ),
    scratch_types=list(_SC_SCRATCH),
    compiler_params=pltpu.CompilerParams(use_tc_tiling_on_sc=False),
)
def _agg12(x, ei, oa, ob, bufa, bufb, idx_s, idx_d, rows, g_sem, s_sem,
           i_sem, d_sem):
    c = lax.axis_index("c")
    s = lax.axis_index("s")
    r0 = s * ROWS_PER_TILE
    col = c * HH
    last = NTILES - 1

    @pl.when(s < last)
    def _():
        pltpu.sync_copy(x.at[pl.ds(r0, ROWS_PER_TILE), pl.ds(col, HH)],
                        bufa.at[pl.ds(r0, ROWS_PER_TILE)])
        pltpu.sync_copy(x.at[pl.ds(r0, ROWS_PER_TILE), pl.ds(col, HH)],
                        bufb.at[pl.ds(r0, ROWS_PER_TILE)])

    @pl.when(s == last)
    def _():
        pltpu.sync_copy(x.at[pl.ds(r0, LAST_ROWS), pl.ds(col, HH)],
                        bufa.at[pl.ds(r0, LAST_ROWS)])
        pltpu.sync_copy(x.at[pl.ds(r0, LAST_ROWS), pl.ds(col, HH)],
                        bufb.at[pl.ds(r0, LAST_ROWS)])

    nch = jnp.where(s == last, NCH_LAST, NCH_FULL)
    e0 = s * E_PT
    plsc.subcore_barrier()
    _edge_pass(ei, bufa, bufb, e0, nch, idx_s, idx_d, rows, g_sem, s_sem,
               i_sem, d_sem)
    plsc.subcore_barrier()
    for k in range(ROWS_PER_TILE // CH):
        rr = r0 + k * CH
        pltpu.sync_copy(bufb.at[pl.ds(rr, CH)], rows.at[0])
        pltpu.sync_copy(rows.at[0], bufa.at[pl.ds(rr, CH)])
    plsc.subcore_barrier()
    _edge_pass(ei, bufb, bufa, e0, nch, idx_s, idx_d, rows, g_sem, s_sem,
               i_sem, d_sem)
    plsc.subcore_barrier()

    @pl.when(c == 0)
    def _():
        pltpu.sync_copy(bufa.at[pl.ds(r0, ROWS_PER_TILE)],
                        oa.at[pl.ds(r0, ROWS_PER_TILE)])

    @pl.when(c == 1)
    def _():
        pltpu.sync_copy(bufa.at[pl.ds(r0, ROWS_PER_TILE)],
                        ob.at[pl.ds(r0, ROWS_PER_TILE)])


@functools.partial(
    pl.kernel,
    mesh=plsc.VectorSubcoreMesh(core_axis_name="c", subcore_axis_name="s"),
    out_type=(
        jax.ShapeDtypeStruct((NP, HH), jnp.float32),
        jax.ShapeDtypeStruct((NP, HH), jnp.float32),
    ),
    scratch_types=list(_SC_SCRATCH),
    compiler_params=pltpu.CompilerParams(use_tc_tiling_on_sc=False),
)
def _agg3(ha, hb, ei, oa, ob, bufa, bufb, idx_s, idx_d, rows, g_sem, s_sem,
          i_sem, d_sem):
    c = lax.axis_index("c")
    s = lax.axis_index("s")
    r0 = s * ROWS_PER_TILE
    last = NTILES - 1

    def stage(table):
        pltpu.sync_copy(table.at[pl.ds(r0, ROWS_PER_TILE)],
                        bufa.at[pl.ds(r0, ROWS_PER_TILE)])
        pltpu.sync_copy(table.at[pl.ds(r0, ROWS_PER_TILE)],
                        bufb.at[pl.ds(r0, ROWS_PER_TILE)])

    @pl.when(c == 0)
    def _():
        stage(ha)

    @pl.when(c == 1)
    def _():
        stage(hb)

    nch = jnp.where(s == last, NCH_LAST, NCH_FULL)
    e0 = s * E_PT
    plsc.subcore_barrier()
    _edge_pass(ei, bufa, bufb, e0, nch, idx_s, idx_d, rows, g_sem, s_sem,
               i_sem, d_sem)
    plsc.subcore_barrier()

    @pl.when(c == 0)
    def _():
        pltpu.sync_copy(bufb.at[pl.ds(r0, ROWS_PER_TILE)],
                        oa.at[pl.ds(r0, ROWS_PER_TILE)])

    @pl.when(c == 1)
    def _():
        pltpu.sync_copy(bufb.at[pl.ds(r0, ROWS_PER_TILE)],
                        ob.at[pl.ds(r0, ROWS_PER_TILE)])


def _mlp_body(oa_ref, ob_ref, w1_ref, b1_ref, pa_ref, pb_ref):
    z = (jnp.dot(oa_ref[...], w1_ref[:HH, :],
                 preferred_element_type=jnp.float32)
         + jnp.dot(ob_ref[...], w1_ref[HH:, :],
                   preferred_element_type=jnp.float32)
         + b1_ref[...])
    act = _SELU_SCALE * jnp.where(z > 0, z, _SELU_ALPHA * (jnp.exp(z) - 1.0))
    pa_ref[...] = act[:, :HH]
    pb_ref[...] = act[:, HH:]


def _mlp(oa, ob, w1, b1):
    br = NP
    grid = (NP // br,)
    return pl.pallas_call(
        _mlp_body,
        grid=grid,
        in_specs=[
            pl.BlockSpec((br, HH), lambda i: (i, 0)),
            pl.BlockSpec((br, HH), lambda i: (i, 0)),
            pl.BlockSpec((D, D), lambda i: (0, 0)),
            pl.BlockSpec((1, D), lambda i: (0, 0)),
        ],
        out_specs=[
            pl.BlockSpec((br, HH), lambda i: (i, 0)),
            pl.BlockSpec((br, HH), lambda i: (i, 0)),
        ],
        out_shape=[
            jax.ShapeDtypeStruct((NP, HH), jnp.float32),
            jax.ShapeDtypeStruct((NP, HH), jnp.float32),
        ],
    )(oa, ob, w1, b1)


def _out_body(qa_ref, qb_ref, w2_ref, b2_ref, o_ref):
    z = (jnp.dot(qa_ref[...], w2_ref[:HH, :],
                 preferred_element_type=jnp.float32)
         + jnp.dot(qb_ref[...], w2_ref[HH:, :],
                   preferred_element_type=jnp.float32)
         + b2_ref[...])
    m = jnp.max(z, axis=1, keepdims=True)
    lse = jnp.log(jnp.sum(jnp.exp(z - m), axis=1, keepdims=True)) + m
    o_ref[...] = z - lse


def _outk(qa, qb, w2, b2):
    br = N
    grid = (N // br,)
    return pl.pallas_call(
        _out_body,
        grid=grid,
        in_specs=[
            pl.BlockSpec((br, HH), lambda i: (i, 0)),
            pl.BlockSpec((br, HH), lambda i: (i, 0)),
            pl.BlockSpec((D, D), lambda i: (0, 0)),
            pl.BlockSpec((1, D), lambda i: (0, 0)),
        ],
        out_specs=pl.BlockSpec((br, D), lambda i: (i, 0)),
        out_shape=jax.ShapeDtypeStruct((N, D), jnp.float32),
    )(qa, qb, w2, b2)


def kernel(x, edge_index, W1, b1, W2, b2):
    h2a, h2b = _agg12(x, edge_index)
    h3a, h3b = _mlp(h2a, h2b, W1, b1.reshape(1, D))
    h4a, h4b = _agg3(h3a, h3b, edge_index)
    return _outk(h4a, h4b, W2, b2.reshape(1, D))


# speedup vs baseline: 1.2329x; 1.0013x over previous
import functools

import jax
import jax.numpy as jnp
from jax import lax
from jax.experimental import pallas as pl
from jax.experimental.pallas import tpu as pltpu
from jax.experimental.pallas import tpu_sc as plsc

N = 10000
E = 320000
D = 128
HH = 64
NP = 10240
NTILES = 16
ROWS_PER_TILE = NP // NTILES
LAST_ROWS = N - (NTILES - 1) * ROWS_PER_TILE
CH = 128
NCH_FULL = 157
E_PT = NCH_FULL * CH
NCH_LAST = (E - (NTILES - 1) * E_PT) // CH
RB = 4
QB = 8
GA = 3
IA = 6

_SELU_ALPHA = 1.6732632423543772
_SELU_SCALE = 1.0507009873554805

_SC_SCRATCH = [
    pltpu.VMEM_SHARED((NP, HH), jnp.float32),
    pltpu.VMEM_SHARED((NP, HH), jnp.float32),
    pltpu.VMEM((QB, CH), jnp.int32),
    pltpu.VMEM((QB, CH), jnp.int32),
    pltpu.VMEM((RB, CH, HH), jnp.float32),
    pltpu.SemaphoreType.DMA((RB,)),
    pltpu.SemaphoreType.DMA((RB,)),
    pltpu.SemaphoreType.DMA((QB,)),
    pltpu.SemaphoreType.DMA((QB,)),
]


def _edge_pass(ei, tb, ab, e0, nch, idx_s, idx_d, rows, g_sem, s_sem,
               i_sem, d_sem):
    for k in range(IA):
        pltpu.async_copy(ei.at[0, pl.ds(e0 + k * CH, CH)], idx_s.at[k],
                         i_sem.at[k])
        pltpu.async_copy(ei.at[1, pl.ds(e0 + k * CH, CH)], idx_d.at[k],
                         d_sem.at[k])
    for k in range(GA):
        pltpu.make_async_copy(ei.at[0, pl.ds(0, CH)], idx_s.at[k],
                              i_sem.at[k]).wait()
        pltpu.async_copy(tb.at[idx_s.at[k]], rows.at[k], g_sem.at[k])

    def step(j, carry):
        u = lax.rem(j, QB)
        b = lax.rem(j, RB)
        pltpu.make_async_copy(tb.at[idx_s.at[0]], rows.at[b],
                              g_sem.at[b]).wait()
        pltpu.make_async_copy(ei.at[1, pl.ds(0, CH)], idx_d.at[u],
                              d_sem.at[u]).wait()
        pltpu.async_copy(rows.at[b], ab.at[idx_d.at[u]], s_sem.at[b],
                         add=True)

        jp = j + GA

        @pl.when(jp < nch)
        def _():
            up = lax.rem(jp, QB)
            bp = lax.rem(jp, RB)

            @pl.when(jp >= RB)
            def _():
                pltpu.make_async_copy(rows.at[bp], ab.at[idx_d.at[up]],
                                      s_sem.at[bp]).wait()

            pltpu.make_async_copy(ei.at[0, pl.ds(0, CH)], idx_s.at[up],
                                  i_sem.at[up]).wait()
            pltpu.async_copy(tb.at[idx_s.at[up]], rows.at[bp], g_sem.at[bp])

        ji = j + IA

        @pl.when(ji < nch)
        def _():
            ui = lax.rem(ji, QB)
            pltpu.async_copy(ei.at[0, pl.ds(e0 + ji * CH, CH)],
                             idx_s.at[ui], i_sem.at[ui])
            pltpu.async_copy(ei.at[1, pl.ds(e0 + ji * CH, CH)],
                             idx_d.at[ui], d_sem.at[ui])
        return carry

    lax.fori_loop(0, nch, step, 0)

    for i in range(RB):
        k = nch - RB + i
        pltpu.make_async_copy(rows.at[lax.rem(k, RB)],
                              ab.at[idx_d.at[lax.rem(k, QB)]],
                              s_sem.at[lax.rem(k, RB)]).wait()


@functools.partial(
    pl.kernel,
    mesh=plsc.VectorSubcoreMesh(core_axis_name="c", subcore_axis_name="s"),
    out_type=(
        jax.ShapeDtypeStruct((NP, HH), jnp.float32),
        jax.ShapeDtypeStruct((NP, HH), jnp.float32),
    ),
    scratch_types=list(_SC_SCRATCH),
    compiler_params=pltpu.CompilerParams(use_tc_tiling_on_sc=False),
)
def _agg12(x, ei, oa, ob, bufa, bufb, idx_s, idx_d, rows, g_sem, s_sem,
           i_sem, d_sem):
    c = lax.axis_index("c")
    s = lax.axis_index("s")
    r0 = s * ROWS_PER_TILE
    col = c * HH
    last = NTILES - 1

    @pl.when(s < last)
    def _():
        pltpu.sync_copy(x.at[pl.ds(r0, ROWS_PER_TILE), pl.ds(col, HH)],
                        bufa.at[pl.ds(r0, ROWS_PER_TILE)])
        pltpu.sync_copy(x.at[pl.ds(r0, ROWS_PER_TILE), pl.ds(col, HH)],
                        bufb.at[pl.ds(r0, ROWS_PER_TILE)])

    @pl.when(s == last)
    def _():
        pltpu.sync_copy(x.at[pl.ds(r0, LAST_ROWS), pl.ds(col, HH)],
                        bufa.at[pl.ds(r0, LAST_ROWS)])
        pltpu.sync_copy(x.at[pl.ds(r0, LAST_ROWS), pl.ds(col, HH)],
                        bufb.at[pl.ds(r0, LAST_ROWS)])

    nch = jnp.where(s == last, NCH_LAST, NCH_FULL)
    e0 = s * E_PT
    plsc.subcore_barrier()
    _edge_pass(ei, bufa, bufb, e0, nch, idx_s, idx_d, rows, g_sem, s_sem,
               i_sem, d_sem)
    plsc.subcore_barrier()
    for k in range(ROWS_PER_TILE // CH):
        rr = r0 + k * CH
        pltpu.sync_copy(bufb.at[pl.ds(rr, CH)], rows.at[0])
        pltpu.sync_copy(rows.at[0], bufa.at[pl.ds(rr, CH)])
    plsc.subcore_barrier()
    _edge_pass(ei, bufb, bufa, e0, nch, idx_s, idx_d, rows, g_sem, s_sem,
               i_sem, d_sem)
    plsc.subcore_barrier()

    @pl.when(c == 0)
    def _():
        pltpu.sync_copy(bufa.at[pl.ds(r0, ROWS_PER_TILE)],
                        oa.at[pl.ds(r0, ROWS_PER_TILE)])

    @pl.when(c == 1)
    def _():
        pltpu.sync_copy(bufa.at[pl.ds(r0, ROWS_PER_TILE)],
                        ob.at[pl.ds(r0, ROWS_PER_TILE)])


@functools.partial(
    pl.kernel,
    mesh=plsc.VectorSubcoreMesh(core_axis_name="c", subcore_axis_name="s"),
    out_type=(
        jax.ShapeDtypeStruct((NP, HH), jnp.float32),
        jax.ShapeDtypeStruct((NP, HH), jnp.float32),
    ),
    scratch_types=list(_SC_SCRATCH),
    compiler_params=pltpu.CompilerParams(use_tc_tiling_on_sc=False),
)
def _agg3(ha, hb, ei, oa, ob, bufa, bufb, idx_s, idx_d, rows, g_sem, s_sem,
          i_sem, d_sem):
    c = lax.axis_index("c")
    s = lax.axis_index("s")
    r0 = s * ROWS_PER_TILE
    last = NTILES - 1

    def stage(table):
        pltpu.sync_copy(table.at[pl.ds(r0, ROWS_PER_TILE)],
                        bufa.at[pl.ds(r0, ROWS_PER_TILE)])
        pltpu.sync_copy(table.at[pl.ds(r0, ROWS_PER_TILE)],
                        bufb.at[pl.ds(r0, ROWS_PER_TILE)])

    @pl.when(c == 0)
    def _():
        stage(ha)

    @pl.when(c == 1)
    def _():
        stage(hb)

    nch = jnp.where(s == last, NCH_LAST, NCH_FULL)
    e0 = s * E_PT
    plsc.subcore_barrier()
    _edge_pass(ei, bufa, bufb, e0, nch, idx_s, idx_d, rows, g_sem, s_sem,
               i_sem, d_sem)
    plsc.subcore_barrier()

    @pl.when(c == 0)
    def _():
        pltpu.sync_copy(bufb.at[pl.ds(r0, ROWS_PER_TILE)],
                        oa.at[pl.ds(r0, ROWS_PER_TILE)])

    @pl.when(c == 1)
    def _():
        pltpu.sync_copy(bufb.at[pl.ds(r0, ROWS_PER_TILE)],
                        ob.at[pl.ds(r0, ROWS_PER_TILE)])


def _mlp_body(oa_ref, ob_ref, w1_ref, b1_ref, pa_ref, pb_ref):
    z = (jnp.dot(oa_ref[...], w1_ref[:HH, :],
                 preferred_element_type=jnp.float32)
         + jnp.dot(ob_ref[...], w1_ref[HH:, :],
                   preferred_element_type=jnp.float32)
         + b1_ref[...])
    act = _SELU_SCALE * jnp.where(z > 0, z, _SELU_ALPHA * (jnp.exp(z) - 1.0))
    pa_ref[...] = act[:, :HH]
    pb_ref[...] = act[:, HH:]


def _mlp(oa, ob, w1, b1):
    br = NP
    grid = (NP // br,)
    return pl.pallas_call(
        _mlp_body,
        grid=grid,
        in_specs=[
            pl.BlockSpec((br, HH), lambda i: (i, 0)),
            pl.BlockSpec((br, HH), lambda i: (i, 0)),
            pl.BlockSpec((D, D), lambda i: (0, 0)),
            pl.BlockSpec((1, D), lambda i: (0, 0)),
        ],
        out_specs=[
            pl.BlockSpec((br, HH), lambda i: (i, 0)),
            pl.BlockSpec((br, HH), lambda i: (i, 0)),
        ],
        out_shape=[
            jax.ShapeDtypeStruct((NP, HH), jnp.float32),
            jax.ShapeDtypeStruct((NP, HH), jnp.float32),
        ],
    )(oa, ob, w1, b1)


def _out_body(qa_ref, qb_ref, w2_ref, b2_ref, o_ref):
    z = (jnp.dot(qa_ref[...], w2_ref[:HH, :],
                 preferred_element_type=jnp.float32)
         + jnp.dot(qb_ref[...], w2_ref[HH:, :],
                   preferred_element_type=jnp.float32)
         + b2_ref[...])
    m = jnp.max(z, axis=1, keepdims=True)
    lse = jnp.log(jnp.sum(jnp.exp(z - m), axis=1, keepdims=True)) + m
    o_ref[...] = z - lse


def _outk(qa, qb, w2, b2):
    br = N
    grid = (N // br,)
    return pl.pallas_call(
        _out_body,
        grid=grid,
        in_specs=[
            pl.BlockSpec((br, HH), lambda i: (i, 0)),
            pl.BlockSpec((br, HH), lambda i: (i, 0)),
            pl.BlockSpec((D, D), lambda i: (0, 0)),
            pl.BlockSpec((1, D), lambda i: (0, 0)),
        ],
        out_specs=pl.BlockSpec((br, D), lambda i: (i, 0)),
        out_shape=jax.ShapeDtypeStruct((N, D), jnp.float32),
    )(qa, qb, w2, b2)


def kernel(x, edge_index, W1, b1, W2, b2):
    h2a, h2b = _agg12(x, edge_index)
    h3a, h3b = _mlp(h2a, h2b, W1, b1.reshape(1, D))
    h4a, h4b = _agg3(h3a, h3b, edge_index)
    return _outk(h4a, h4b, W2, b2.reshape(1, D))
